# baseline (speedup 1.0000x reference)
import functools

import numpy as np
import jax
import jax.numpy as jnp
from jax import lax
from jax.experimental import pallas as pl
from jax.experimental.pallas import tpu as pltpu


def _bilinear_weights(out_size, in_size):
    a = np.zeros((out_size, in_size), dtype=np.float32)
    scale = in_size / out_size
    for i in range(out_size):
        src = max((i + 0.5) * scale - 0.5, 0.0)
        lo = min(int(src), in_size - 1)
        hi = min(lo + 1, in_size - 1)
        frac = src - lo
        a[i, lo] += 1.0 - frac
        a[i, hi] += frac
    return a


def _upsample_matrix(H, W, Hm, Wm):
    ah = _bilinear_weights(H, Hm)
    aw = _bilinear_weights(W, Wm)
    u = np.einsum("hm,wk->mkhw", ah, aw).reshape(Hm * Wm, H * W)
    return jnp.asarray(u)


def _esa_kernel(x_ref, w1_ref, b1_ref, w2_ref, b2_ref, wm_ref, bm_ref,
                w3_ref, b3_ref, w3p_ref, b3p_ref, wf_ref, bf_ref,
                w4_ref, b4_ref, u_ref, o_ref, c13_ref, tap_ref, tap9_ref,
                *, f, H, W, H2, Hm):
    xb = x_ref[0]

    c1f = jnp.dot(w1_ref[...], xb,
                  preferred_element_type=jnp.float32) + b1_ref[...]

    c13_ref[...] = c1f.reshape(f, H, W)
    jj = lax.broadcasted_iota(jnp.int32, (W, 3 * H2), 0)
    xx = lax.broadcasted_iota(jnp.int32, (W, 3 * H2), 1)
    ck = jnp.where(jj == 2 * (xx % H2) + xx // H2, 1.0, 0.0).astype(jnp.float32)
    for ky in range(3):
        rows = c13_ref[:, pl.ds(ky, H2, 2), :]
        z = lax.dot_general(rows, ck, (((2,), (0,)), ((), ())),
                            precision=lax.Precision.HIGHEST,
                            preferred_element_type=jnp.float32)
        for kx in range(3):
            tap_ref[(3 * ky + kx) * f:(3 * ky + kx + 1) * f, :] = (
                z[:, :, kx * H2:(kx + 1) * H2].reshape(f, H2 * H2))
    c1 = (jnp.dot(w2_ref[...], tap_ref[...],
                  preferred_element_type=jnp.float32)
          + b2_ref[...]).reshape(f, H2, H2)

    cols = [jnp.max(c1[:, :, 3 * i:3 * i + 7], axis=2, keepdims=True)
            for i in range(Hm)]
    cm = jnp.concatenate(cols, axis=2)
    rws = [jnp.max(cm[:, 3 * i:3 * i + 7, :], axis=1, keepdims=True)
           for i in range(Hm)]
    vm = jnp.concatenate(rws, axis=1)

    def conv3x3_same(v, w_ref, b_ref, relu):
        vp = jnp.pad(v, ((0, 0), (1, 1), (1, 1)))
        for t, (ky, kx) in enumerate((ky, kx) for ky in range(3)
                                     for kx in range(3)):
            tap9_ref[t * f:(t + 1) * f, :] = (
                vp[:, ky:ky + Hm, kx:kx + Hm].reshape(f, Hm * Hm))
        y = (jnp.dot(w_ref[...], tap9_ref[...],
                     preferred_element_type=jnp.float32)
             + b_ref[...]).reshape(f, Hm, Hm)
        return jnp.maximum(y, 0.0) if relu else y

    vr = conv3x3_same(vm, wm_ref, bm_ref, True)
    c3 = conv3x3_same(vr, w3_ref, b3_ref, True)
    c3 = conv3x3_same(c3, w3p_ref, b3p_ref, False)

    c3flat = c3.reshape(f, Hm * Hm)
    c3u = jnp.dot(c3flat, u_ref[...], preferred_element_type=jnp.float32)

    cf = jnp.dot(wf_ref[...], c1f,
                 preferred_element_type=jnp.float32) + bf_ref[...]
    c4 = jnp.dot(w4_ref[...], c3u + cf,
                 preferred_element_type=jnp.float32) + b4_ref[...]
    gate = 1.0 / (1.0 + jnp.exp(-c4))
    o_ref[0] = (xb * gate).astype(o_ref.dtype)


def kernel(x, b1, b2, b3, b3_, b4, b_f, b_max, w1, w2, w3, w3_, w4, w_f, w_max):
    N, C, H, W = x.shape
    f = C // 4
    H2 = (H - 3) // 2 + 1
    Hm = (H2 - 7) // 3 + 1
    S = H * W

    def tap_layout(w):
        return jnp.transpose(w, (0, 2, 3, 1)).reshape(w.shape[0], 9 * w.shape[1])

    u = _upsample_matrix(H, W, Hm, Hm)

    col = lambda b: b.reshape(b.shape[0], 1)
    full = lambda shape: pl.BlockSpec(shape, lambda i, j: tuple(0 for _ in shape))

    out = pl.pallas_call(
        functools.partial(_esa_kernel, f=f, H=H, W=W, H2=H2, Hm=Hm),
        out_shape=jax.ShapeDtypeStruct((N, C, S), x.dtype),
        grid=(2, N // 2),
        in_specs=[
            pl.BlockSpec((1, C, S), lambda i, j: (i * (N // 2) + j, 0, 0)),
            full((f, C)), full((f, 1)),
            full((f, 9 * f)), full((f, 1)),
            full((f, 9 * f)), full((f, 1)),
            full((f, 9 * f)), full((f, 1)),
            full((f, 9 * f)), full((f, 1)),
            full((f, f)), full((f, 1)),
            full((C, f)), full((C, 1)),
            full((Hm * Hm, S)),
        ],
        out_specs=pl.BlockSpec((1, C, S), lambda i, j: (i * (N // 2) + j, 0, 0)),
        scratch_shapes=[pltpu.VMEM((f, H, W), jnp.float32),
                        pltpu.VMEM((9 * f, H2 * H2), jnp.float32),
                        pltpu.VMEM((9 * f, Hm * Hm), jnp.float32)],
        compiler_params=pltpu.CompilerParams(
            dimension_semantics=("parallel", "arbitrary"),
            vmem_limit_bytes=100 * 1024 * 1024),
    )(x.reshape(N, C, S),
      w1.reshape(f, C), col(b1),
      tap_layout(w2), col(b2),
      tap_layout(w_max), col(b_max),
      tap_layout(w3), col(b3),
      tap_layout(w3_), col(b3_),
      w_f.reshape(f, f), col(b_f),
      w4.reshape(C, f), col(b4),
      u)
    return out.reshape(N, C, H, W)


# speedup vs baseline: 1.2890x; 1.2890x over previous
import functools

import numpy as np
import jax
import jax.numpy as jnp
from jax import lax
from jax.experimental import pallas as pl
from jax.experimental.pallas import tpu as pltpu


def _bilinear_weights(out_size, in_size):
    a = np.zeros((out_size, in_size), dtype=np.float32)
    scale = in_size / out_size
    for i in range(out_size):
        src = max((i + 0.5) * scale - 0.5, 0.0)
        lo = min(int(src), in_size - 1)
        hi = min(lo + 1, in_size - 1)
        frac = src - lo
        a[i, lo] += 1.0 - frac
        a[i, hi] += frac
    return a


def _upsample_matrix(H, W, Hm, Wm):
    ah = _bilinear_weights(H, Hm)
    aw = _bilinear_weights(W, Wm)
    u = np.einsum("hm,wk->mkhw", ah, aw).reshape(Hm * Wm, H * W)
    return jnp.asarray(u)


def _esa_kernel(x_ref, w1_ref, b1_ref, w2_ref, b2_ref, wm_ref, bm_ref,
                w3_ref, b3_ref, w3p_ref, b3p_ref, wf_ref, bf_ref,
                w4_ref, b4_ref, u_ref, o_ref, xf_ref, c13_ref, tap_ref,
                tap9_ref, *, f, H, W, H2, Hm):
    x3 = x_ref[0]
    xf_ref[...] = x3.reshape(x3.shape[0], H * W)

    c1f = jnp.dot(w1_ref[...], xf_ref[...],
                  preferred_element_type=jnp.float32) + b1_ref[...]

    c13_ref[...] = c1f.reshape(f, H, W)
    jj = lax.broadcasted_iota(jnp.int32, (W, 3 * H2), 0)
    xx = lax.broadcasted_iota(jnp.int32, (W, 3 * H2), 1)
    ck = jnp.where(jj == 2 * (xx % H2) + xx // H2, 1.0, 0.0).astype(jnp.float32)
    for ky in range(3):
        rows = c13_ref[:, pl.ds(ky, H2, 2), :]
        z = lax.dot_general(rows, ck, (((2,), (0,)), ((), ())),
                            precision=lax.Precision.HIGHEST,
                            preferred_element_type=jnp.float32)
        for kx in range(3):
            tap_ref[(3 * ky + kx) * f:(3 * ky + kx + 1) * f, :] = (
                z[:, :, kx * H2:(kx + 1) * H2].reshape(f, H2 * H2))
    c1 = (jnp.dot(w2_ref[...], tap_ref[...],
                  preferred_element_type=jnp.float32)
          + b2_ref[...]).reshape(f, H2, H2)

    cols = [jnp.max(c1[:, :, 3 * i:3 * i + 7], axis=2, keepdims=True)
            for i in range(Hm)]
    cm = jnp.concatenate(cols, axis=2)
    rws = [jnp.max(cm[:, 3 * i:3 * i + 7, :], axis=1, keepdims=True)
           for i in range(Hm)]
    vm = jnp.concatenate(rws, axis=1)

    def conv3x3_same(v, w_ref, b_ref, relu):
        vp = jnp.pad(v, ((0, 0), (1, 1), (1, 1)))
        for t, (ky, kx) in enumerate((ky, kx) for ky in range(3)
                                     for kx in range(3)):
            tap9_ref[t * f:(t + 1) * f, :] = (
                vp[:, ky:ky + Hm, kx:kx + Hm].reshape(f, Hm * Hm))
        y = (jnp.dot(w_ref[...], tap9_ref[...],
                     preferred_element_type=jnp.float32)
             + b_ref[...]).reshape(f, Hm, Hm)
        return jnp.maximum(y, 0.0) if relu else y

    vr = conv3x3_same(vm, wm_ref, bm_ref, True)
    c3 = conv3x3_same(vr, w3_ref, b3_ref, True)
    c3 = conv3x3_same(c3, w3p_ref, b3p_ref, False)

    c3u = jnp.dot(c3.reshape(f, Hm * Hm), u_ref[...],
                  preferred_element_type=jnp.float32)

    cf = jnp.dot(wf_ref[...], c1f,
                 preferred_element_type=jnp.float32) + bf_ref[...]
    c13_ref[...] = (c3u + cf).reshape(f, H, W)
    c4 = lax.dot_general(w4_ref[...], c13_ref[...], (((1,), (0,)), ((), ())),
                         preferred_element_type=jnp.float32) + b4_ref[...]
    gate = 1.0 / (1.0 + jnp.exp(-c4))
    o_ref[0] = (x3 * gate).astype(o_ref.dtype)


def kernel(x, b1, b2, b3, b3_, b4, b_f, b_max, w1, w2, w3, w3_, w4, w_f, w_max):
    N, C, H, W = x.shape
    f = C // 4
    H2 = (H - 3) // 2 + 1
    Hm = (H2 - 7) // 3 + 1
    S = H * W

    def tap_layout(w):
        return jnp.transpose(w, (0, 2, 3, 1)).reshape(w.shape[0], 9 * w.shape[1])

    u = _upsample_matrix(H, W, Hm, Hm)

    col = lambda b: b.reshape(b.shape[0], 1)
    full = lambda shape: pl.BlockSpec(shape, lambda n: tuple(0 for _ in shape))

    return pl.pallas_call(
        functools.partial(_esa_kernel, f=f, H=H, W=W, H2=H2, Hm=Hm),
        out_shape=jax.ShapeDtypeStruct((N, C, H, W), x.dtype),
        grid=(N,),
        in_specs=[
            pl.BlockSpec((1, C, H, W), lambda n: (n, 0, 0, 0)),
            full((f, C)), full((f, 1)),
            full((f, 9 * f)), full((f, 1)),
            full((f, 9 * f)), full((f, 1)),
            full((f, 9 * f)), full((f, 1)),
            full((f, 9 * f)), full((f, 1)),
            full((f, f)), full((f, 1)),
            full((C, f)), full((C, 1, 1)),
            full((Hm * Hm, S)),
        ],
        out_specs=pl.BlockSpec((1, C, H, W), lambda n: (n, 0, 0, 0)),
        scratch_shapes=[pltpu.VMEM((C, S), jnp.float32),
                        pltpu.VMEM((f, H, W), jnp.float32),
                        pltpu.VMEM((9 * f, H2 * H2), jnp.float32),
                        pltpu.VMEM((9 * f, Hm * Hm), jnp.float32)],
        compiler_params=pltpu.CompilerParams(
            dimension_semantics=("parallel",),
            vmem_limit_bytes=100 * 1024 * 1024),
    )(x,
      w1.reshape(f, C), col(b1),
      tap_layout(w2), col(b2),
      tap_layout(w_max), col(b_max),
      tap_layout(w3), col(b3),
      tap_layout(w3_), col(b3_),
      w_f.reshape(f, f), col(b_f),
      w4.reshape(C, f), b4.reshape(C, 1, 1),
      u)


# speedup vs baseline: 1.3824x; 1.0725x over previous
import functools

import numpy as np
import jax
import jax.numpy as jnp
from jax import lax
from jax.experimental import pallas as pl
from jax.experimental.pallas import tpu as pltpu


def _bilinear_weights(out_size, in_size):
    a = np.zeros((out_size, in_size), dtype=np.float32)
    scale = in_size / out_size
    for i in range(out_size):
        src = max((i + 0.5) * scale - 0.5, 0.0)
        lo = min(int(src), in_size - 1)
        hi = min(lo + 1, in_size - 1)
        frac = src - lo
        a[i, lo] += 1.0 - frac
        a[i, hi] += frac
    return a


def _upsample_matrix(H, W, Hm, Wm):
    ah = _bilinear_weights(H, Hm)
    aw = _bilinear_weights(W, Wm)
    u = np.einsum("hm,wk->mkhw", ah, aw).reshape(Hm * Wm, H * W)
    return jnp.asarray(u)


def _colsel_matrix(W, H2):
    ck = np.zeros((W, 3 * H2), dtype=np.float32)
    for kx in range(3):
        for x in range(H2):
            ck[2 * x + kx, kx * H2 + x] = 1.0
    return jnp.asarray(ck)


def _pick3_matrix(L, Hm):
    p = np.zeros((L, Hm), dtype=np.float32)
    for i in range(Hm):
        p[3 * i, i] = 1.0
    return jnp.asarray(p)


def _esa_kernel(x_ref, w1_ref, w2_ref, wm_ref, w3_ref, w3p_ref, wf_ref,
                w4_ref, bb_ref, ck_ref, p3_ref, u_ref, o_ref, xf_ref,
                c13_ref, z_ref, tap_ref, tap9_ref, mp_ref, *, f, H, W, H2, Hm):
    x3 = x_ref[0]
    xf_ref[...] = x3.reshape(x3.shape[0], H * W)

    c1f = jnp.dot(w1_ref[...], xf_ref[...],
                  preferred_element_type=jnp.float32) + bb_ref[0:f, 0:1]

    c13_ref[...] = c1f.reshape(f, H, W)
    z = jnp.dot(c13_ref[...].reshape(f * H, W), ck_ref[...],
                precision=lax.Precision.HIGHEST,
                preferred_element_type=jnp.float32)
    z_ref[...] = z.reshape(f, H, 3 * H2)
    for ky in range(3):
        zky = z_ref[:, pl.ds(ky, H2, 2), :]
        for kx in range(3):
            tap_ref[(3 * ky + kx) * f:(3 * ky + kx + 1) * f, :] = (
                zky[:, :, kx * H2:(kx + 1) * H2].reshape(f, H2 * H2))
    c1 = (jnp.dot(w2_ref[...], tap_ref[...],
                  preferred_element_type=jnp.float32)
          + bb_ref[0:f, 1:2]).reshape(f, H2, H2)

    L = 3 * (Hm - 1) + 1
    w2l = jnp.maximum(c1[:, :, 0:H2 - 1], c1[:, :, 1:H2])
    w4l = jnp.maximum(w2l[:, :, 0:H2 - 3], w2l[:, :, 2:H2 - 1])
    w7l = jnp.maximum(w4l[:, :, 0:L], w4l[:, :, 3:L + 3])
    cm = jnp.dot(w7l.reshape(f * H2, L), p3_ref[...],
                 precision=lax.Precision.HIGHEST,
                 preferred_element_type=jnp.float32).reshape(f, H2, Hm)
    r2 = jnp.maximum(cm[:, 0:H2 - 1, :], cm[:, 1:H2, :])
    r4 = jnp.maximum(r2[:, 0:H2 - 3, :], r2[:, 2:H2 - 1, :])
    mp_ref[...] = jnp.maximum(r4[:, 0:L, :], r4[:, 3:L + 3, :])
    vm = mp_ref[:, pl.ds(0, Hm, 3), :]

    def conv3x3_same(v, w_ref, bcol, relu):
        vp = jnp.pad(v, ((0, 0), (1, 1), (1, 1)))
        for t, (ky, kx) in enumerate((ky, kx) for ky in range(3)
                                     for kx in range(3)):
            tap9_ref[t * f:(t + 1) * f, :] = (
                vp[:, ky:ky + Hm, kx:kx + Hm].reshape(f, Hm * Hm))
        y = (jnp.dot(w_ref[...], tap9_ref[...],
                     preferred_element_type=jnp.float32)
             + bb_ref[0:f, bcol:bcol + 1]).reshape(f, Hm, Hm)
        return jnp.maximum(y, 0.0) if relu else y

    vr = conv3x3_same(vm, wm_ref, 2, True)
    c3 = conv3x3_same(vr, w3_ref, 3, True)
    c3 = conv3x3_same(c3, w3p_ref, 4, False)

    c3u = jnp.dot(c3.reshape(f, Hm * Hm), u_ref[...],
                  preferred_element_type=jnp.float32)

    cf = jnp.dot(wf_ref[...], c1f,
                 preferred_element_type=jnp.float32) + bb_ref[0:f, 5:6]
    c4 = jnp.dot(w4_ref[...], c3u + cf,
                 preferred_element_type=jnp.float32) + bb_ref[:, 6:7]
    gate = 1.0 / (1.0 + jnp.exp(-c4))
    o_ref[0] = (xf_ref[...] * gate).reshape(o_ref.shape[1:]).astype(o_ref.dtype)


def kernel(x, b1, b2, b3, b3_, b4, b_f, b_max, w1, w2, w3, w3_, w4, w_f, w_max):
    N, C, H, W = x.shape
    f = C // 4
    H2 = (H - 3) // 2 + 1
    Hm = (H2 - 7) // 3 + 1
    S = H * W

    def tap_layout(w):
        return jnp.transpose(w, (0, 2, 3, 1)).reshape(w.shape[0], 9 * w.shape[1])

    u = _upsample_matrix(H, W, Hm, Hm)
    ck = _colsel_matrix(W, H2)
    p3 = _pick3_matrix(3 * (Hm - 1) + 1, Hm)
    pad = lambda b: jnp.pad(b, (0, C - b.shape[0]))
    bb = jnp.stack([pad(b1), pad(b2), pad(b_max), pad(b3), pad(b3_),
                    pad(b_f), b4], axis=1)

    full = lambda shape: pl.BlockSpec(shape, lambda n: tuple(0 for _ in shape))

    return pl.pallas_call(
        functools.partial(_esa_kernel, f=f, H=H, W=W, H2=H2, Hm=Hm),
        out_shape=jax.ShapeDtypeStruct((N, C, H, W), x.dtype),
        grid=(N,),
        in_specs=[
            pl.BlockSpec((1, C, H, W), lambda n: (n, 0, 0, 0)),
            full((f, C)),
            full((f, 9 * f)),
            full((f, 9 * f)),
            full((f, 9 * f)),
            full((f, 9 * f)),
            full((f, f)),
            full((C, f)),
            full((C, 7)),
            full((W, 3 * H2)),
            full((3 * (Hm - 1) + 1, Hm)),
            full((Hm * Hm, S)),
        ],
        out_specs=pl.BlockSpec((1, C, H, W), lambda n: (n, 0, 0, 0)),
        scratch_shapes=[pltpu.VMEM((C, S), jnp.float32),
                        pltpu.VMEM((f, H, W), jnp.float32),
                        pltpu.VMEM((f, H, 3 * H2), jnp.float32),
                        pltpu.VMEM((9 * f, H2 * H2), jnp.float32),
                        pltpu.VMEM((9 * f, Hm * Hm), jnp.float32),
                        pltpu.VMEM((f, 3 * (Hm - 1) + 1, Hm), jnp.float32)],
        compiler_params=pltpu.CompilerParams(
            dimension_semantics=("parallel",),
            vmem_limit_bytes=100 * 1024 * 1024),
    )(x,
      w1.reshape(f, C),
      tap_layout(w2),
      tap_layout(w_max),
      tap_layout(w3),
      tap_layout(w3_),
      w_f.reshape(f, f),
      w4.reshape(C, f),
      bb, ck, p3, u)
